# baseline (speedup 1.0000x reference)
import functools

import jax
import jax.numpy as jnp
from jax import lax
from jax.experimental import pallas as pl
from jax.experimental.pallas import tpu as pltpu
from jax.experimental.pallas import tpu_sc as plsc

NC = 2
NS = 16
NW = NC * NS

T = 16384
S = 50
D = 32
V = 1000000

TT = T // 128
TPW = TT // NW
NBLK = TPW * S

_mesh = plsc.VectorSubcoreMesh(core_axis_name="c", subcore_axis_name="s")


@functools.partial(
    pl.kernel,
    mesh=_mesh,
    compiler_params=pltpu.CompilerParams(
        use_tc_tiling_on_sc=False, needs_layout_passes=False),
    out_type=jax.ShapeDtypeStruct((S, D // 8, TT, 8, 128), jnp.float32),
    scratch_types=[
        pltpu.VMEM((TPW, S * 128), jnp.int32),
        pltpu.VMEM((5, 128, D), jnp.float32),
        pltpu.VMEM((5, D // 8, 8, 129), jnp.float32),
        pltpu.SemaphoreType.DMA,
        pltpu.SemaphoreType.DMA,
    ],
)
def _gather_kernel(ids_hbm, table_hbm, out_hbm, ids_v, rows_v, stg_v,
                   gsem, ssem):
    wid = lax.axis_index("s") * NC + lax.axis_index("c")
    gtt0 = wid * TPW
    pltpu.sync_copy(ids_hbm.at[pl.ds(gtt0, TPW)], ids_v)
    iota = lax.iota(jnp.int32, 16)
    e4a = iota // 8
    e8v = iota - e4a * 8
    e4b = e4a + 2

    def start_gather(it, b):
        tl = it // S
        s = it - tl * S
        pltpu.async_copy(
            table_hbm.at[ids_v.at[tl, pl.ds(s * 128, 128)]],
            rows_v.at[b], gsem)

    def wait_gather(b):
        pltpu.make_async_copy(
            table_hbm.at[ids_v.at[0, pl.ds(0, 128)]], rows_v.at[b],
            gsem).wait()

    def start_store(it, b):
        tl = it // S
        s = it - tl * S
        pltpu.async_copy(stg_v.at[b, :, :, pl.ds(0, 128)],
                         out_hbm.at[s, :, gtt0 + tl], ssem)

    def wait_store(b):
        pltpu.make_async_copy(stg_v.at[b, :, :, pl.ds(0, 128)],
                              out_hbm.at[0, :, gtt0], ssem).wait()

    def transpose_block(b):
        rows = rows_v.at[b]
        stg = stg_v.at[b]
        vals = {}
        for t in range(130):
            if t < 128:
                vals[t] = (rows[t, pl.ds(0, 16)], rows[t, pl.ds(16, 16)],
                           jnp.full((16,), t, jnp.int32))
            if t >= 2:
                v0, v1, tv = vals.pop(t - 2)
                plsc.store_scatter(stg, [e4a, e8v, tv], v0)
                plsc.store_scatter(stg, [e4b, e8v, tv], v1)

    for b in range(5):
        start_gather(b, b)

    def body(j, carry):
        for b in range(5):
            it = j * 5 + b
            wait_gather(b)

            @pl.when(it >= 5)
            def _():
                wait_store(b)

            transpose_block(b)
            start_store(it, b)

            @pl.when(it < NBLK - 5)
            def _():
                start_gather(it + 5, b)

        return carry

    lax.fori_loop(0, NBLK // 5, body, 0)
    for b in range(5):
        wait_store(b)


def kernel(token_ids, weight):
    ids2 = (token_ids.astype(jnp.int32).T
            .reshape(S, TT, 128).transpose(1, 0, 2).reshape(TT, S * 128))
    out5 = _gather_kernel(ids2, weight)
    return out5.transpose(2, 4, 0, 1, 3).reshape(T, S, D)


# speedup vs baseline: 1.0077x; 1.0077x over previous
import functools

import jax
import jax.numpy as jnp
from jax import lax
from jax.experimental import pallas as pl
from jax.experimental.pallas import tpu as pltpu
from jax.experimental.pallas import tpu_sc as plsc

NC = 2
NS = 16
NW = NC * NS

T = 16384
S = 50
D = 32
V = 1000000

TT = T // 128
TPW = TT // NW
NBLK = TPW * S

_mesh = plsc.VectorSubcoreMesh(core_axis_name="c", subcore_axis_name="s")


@functools.partial(
    pl.kernel,
    mesh=_mesh,
    compiler_params=pltpu.CompilerParams(
        use_tc_tiling_on_sc=False, needs_layout_passes=False),
    out_type=jax.ShapeDtypeStruct((S, D // 8, TT, 8, 128), jnp.float32),
    scratch_types=[
        pltpu.VMEM((TPW, S * 128), jnp.int32),
        pltpu.VMEM((4, 128, D), jnp.float32),
        pltpu.VMEM((4, D // 8, 8, 129), jnp.float32),
        pltpu.SemaphoreType.DMA,
        pltpu.SemaphoreType.DMA,
    ],
)
def _gather_kernel(ids_hbm, table_hbm, out_hbm, ids_v, rows_v, stg_v,
                   gsem, ssem):
    wid = lax.axis_index("s") * NC + lax.axis_index("c")
    gtt0 = wid * TPW
    pltpu.sync_copy(ids_hbm.at[pl.ds(gtt0, TPW)], ids_v)
    iota = lax.iota(jnp.int32, 16)
    e4a = iota // 8
    e8v = iota - e4a * 8
    e4b = e4a + 2

    def start_gather(it, b):
        tl = it // S
        s = it - tl * S
        pltpu.async_copy(
            table_hbm.at[ids_v.at[tl, pl.ds(s * 128, 128)]],
            rows_v.at[b], gsem)

    def wait_gather(b):
        pltpu.make_async_copy(
            table_hbm.at[ids_v.at[0, pl.ds(0, 128)]], rows_v.at[b],
            gsem).wait()

    def start_store(it, b):
        tl = it // S
        s = it - tl * S
        pltpu.async_copy(stg_v.at[b, :, :, pl.ds(0, 128)],
                         out_hbm.at[s, :, gtt0 + tl], ssem)

    def wait_store(b):
        pltpu.make_async_copy(stg_v.at[b, :, :, pl.ds(0, 128)],
                              out_hbm.at[0, :, gtt0], ssem).wait()

    def transpose_block(b):
        rows = rows_v.at[b]
        stg = stg_v.at[b]
        vals = {}
        for t in range(130):
            if t < 128:
                vals[t] = (rows[t, pl.ds(0, 16)], rows[t, pl.ds(16, 16)],
                           jnp.full((16,), t, jnp.int32))
            if t >= 2:
                v0, v1, tv = vals.pop(t - 2)
                plsc.store_scatter(stg, [e4a, e8v, tv], v0)
                plsc.store_scatter(stg, [e4b, e8v, tv], v1)

    for b in range(4):
        start_gather(b, b)

    def body(j, carry):
        for b in range(4):
            it = j * 4 + b
            wait_gather(b)

            @pl.when(it >= 4)
            def _():
                wait_store(b)

            transpose_block(b)
            start_store(it, b)

            @pl.when(it < NBLK - 4)
            def _():
                start_gather(it + 4, b)

        return carry

    lax.fori_loop(0, NBLK // 4, body, 0)
    for b in range(4):
        wait_store(b)


def kernel(token_ids, weight):
    ids2 = (token_ids.astype(jnp.int32).T
            .reshape(S, TT, 128).transpose(1, 0, 2).reshape(TT, S * 128))
    out5 = _gather_kernel(ids2, weight)
    return out5.transpose(2, 4, 0, 1, 3).reshape(T, S, D)
